# Initial kernel scaffold; baseline (speedup 1.0000x reference)
import jax
import jax.numpy as jnp
from jax.experimental import pallas as pl


def kernel(x, pos_embed_weight):
    raise NotImplementedError("write your pallas kernel here")


# speedup vs baseline: 2.1821x; 2.1821x over previous
import jax
import jax.numpy as jnp
from jax.experimental import pallas as pl


_S_BLK = 256


def _pe_add_kernel(x_ref, w_ref, o_ref):
    o_ref[...] = x_ref[...] + w_ref[...][:, None, :]


def kernel(x, pos_embed_weight):
    seq_len, batch, d_model = x.shape
    grid = (seq_len // _S_BLK,)
    return pl.pallas_call(
        _pe_add_kernel,
        grid=grid,
        in_specs=[
            pl.BlockSpec((_S_BLK, batch, d_model), lambda i: (i, 0, 0)),
            pl.BlockSpec((_S_BLK, d_model), lambda i: (i, 0)),
        ],
        out_specs=pl.BlockSpec((_S_BLK, batch, d_model), lambda i: (i, 0, 0)),
        out_shape=jax.ShapeDtypeStruct(x.shape, x.dtype),
    )(x, pos_embed_weight)
